# baseline (speedup 1.0000x reference)
import functools

import jax
import jax.numpy as jnp
from jax import lax
from jax.experimental import pallas as pl
from jax.experimental.pallas import tpu as pltpu
from jax.experimental.pallas import tpu_sc as plsc

_B = 16384
_F = 64

_NS = plsc.get_sparse_core_info().num_subcores
_NW = 2 * _NS
_BPW = _B // _NW
_CHUNK = 128
_NCHUNK = _BPW // _CHUNK


def _sc_gather(idx_hbm, mu_hbm, out_hbm, idx_v, mu_v, sem):
    wid = lax.axis_index("s") * 2 + lax.axis_index("c")
    base = wid * _BPW
    pltpu.sync_copy(idx_hbm.at[pl.ds(wid * _NCHUNK, _NCHUNK)], idx_v)
    copies = [
        pltpu.async_copy(mu_hbm.at[idx_v.at[j]],
                         mu_v.at[pl.ds(j * _CHUNK, _CHUNK)], sem)
        for j in range(_NCHUNK)
    ]
    for c in copies:
        c.wait()
    pltpu.sync_copy(mu_v, out_hbm.at[pl.ds(base, _BPW)])


def _matvec_body(ut_ref, vt_ref, wu_ref, wi_ref, o_ref):
    ut = ut_ref[...]
    vt = vt_ref[...]
    wu = wu_ref[...]
    wi = wi_ref[...]
    s = (jnp.dot(wu, ut, preferred_element_type=jnp.float32)
         + jnp.dot(wi, vt, preferred_element_type=jnp.float32))
    o_ref[...] = s.reshape(o_ref.shape)


def _combine_body(lg_ref, mu_ref, eps_ref, o_ref):
    o_ref[...] = lg_ref[...] + jnp.exp(mu_ref[...] + eps_ref[...])


def kernel(users, items, items_idx, eps, W_user, W_item, mu):
    idx2d = items_idx.reshape(_B // _CHUNK, _CHUNK)

    mesh = plsc.VectorSubcoreMesh(core_axis_name="c", subcore_axis_name="s")
    sc = functools.partial(
        pl.kernel,
        mesh=mesh,
        out_type=jax.ShapeDtypeStruct((_B,), jnp.float32),
        scratch_types=[
            pltpu.VMEM((_NCHUNK, _CHUNK), jnp.int32),
            pltpu.VMEM((_BPW,), jnp.float32),
            pltpu.SemaphoreType.DMA,
        ],
    )(_sc_gather)
    mu_g = sc(idx2d, mu)

    cols = 4096
    logits = pl.pallas_call(
        _matvec_body,
        grid=(_B // cols,),
        in_specs=[
            pl.BlockSpec((_F, cols), lambda i: (0, i)),
            pl.BlockSpec((_F, cols), lambda i: (0, i)),
            pl.BlockSpec((1, _F), lambda i: (0, 0)),
            pl.BlockSpec((1, _F), lambda i: (0, 0)),
        ],
        out_specs=pl.BlockSpec((cols,), lambda i: (i,)),
        out_shape=jax.ShapeDtypeStruct((_B,), jnp.float32),
    )(users.T, items.T, W_user.T, W_item.T)

    return pl.pallas_call(
        _combine_body,
        out_shape=jax.ShapeDtypeStruct((_B,), jnp.float32),
    )(logits, mu_g, eps)


# speedup vs baseline: 1.0290x; 1.0290x over previous
import functools

import jax
import jax.numpy as jnp
from jax import lax
from jax.experimental import pallas as pl
from jax.experimental.pallas import tpu as pltpu
from jax.experimental.pallas import tpu_sc as plsc

_B = 16384
_F = 64

_NS = plsc.get_sparse_core_info().num_subcores
_NW = _NS
_BPW = _B // _NW
_CHUNK = 128
_NCHUNK = _BPW // _CHUNK


def _sc_gather(idx_hbm, mu_hbm, out_hbm, idx_v, mu_v, sem):
    wid = lax.axis_index("s")
    base = wid * _BPW
    pltpu.sync_copy(idx_hbm.at[pl.ds(wid * _NCHUNK, _NCHUNK)], idx_v)
    copies = [
        pltpu.async_copy(mu_hbm.at[idx_v.at[j]],
                         mu_v.at[pl.ds(j * _CHUNK, _CHUNK)], sem)
        for j in range(_NCHUNK)
    ]
    for c in copies:
        c.wait()
    pltpu.sync_copy(mu_v, out_hbm.at[pl.ds(base, _BPW)])


def _matvec_body(ut_ref, vt_ref, wu_ref, wi_ref, o_ref):
    ut = ut_ref[...]
    vt = vt_ref[...]
    wu = wu_ref[...]
    wi = wi_ref[...]
    s = (jnp.dot(wu, ut, preferred_element_type=jnp.float32)
         + jnp.dot(wi, vt, preferred_element_type=jnp.float32))
    o_ref[...] = s.reshape(o_ref.shape)


def _combine_body(lg_ref, mu_ref, eps_ref, o_ref):
    o_ref[...] = lg_ref[...] + jnp.exp(mu_ref[...] + eps_ref[...])


def kernel(users, items, items_idx, eps, W_user, W_item, mu):
    idx2d = items_idx.reshape(_B // _CHUNK, _CHUNK)

    mesh = plsc.VectorSubcoreMesh(core_axis_name="c", subcore_axis_name="s",
                                  num_cores=1)
    sc = functools.partial(
        pl.kernel,
        mesh=mesh,
        out_type=jax.ShapeDtypeStruct((_B,), jnp.float32),
        scratch_types=[
            pltpu.VMEM((_NCHUNK, _CHUNK), jnp.int32),
            pltpu.VMEM((_BPW,), jnp.float32),
            pltpu.SemaphoreType.DMA,
        ],
    )(_sc_gather)
    mu_g = sc(idx2d, mu)

    cols = 4096
    logits = pl.pallas_call(
        _matvec_body,
        grid=(_B // cols,),
        in_specs=[
            pl.BlockSpec((_F, cols), lambda i: (0, i)),
            pl.BlockSpec((_F, cols), lambda i: (0, i)),
            pl.BlockSpec((1, _F), lambda i: (0, 0)),
            pl.BlockSpec((1, _F), lambda i: (0, 0)),
        ],
        out_specs=pl.BlockSpec((cols,), lambda i: (i,)),
        out_shape=jax.ShapeDtypeStruct((_B,), jnp.float32),
    )(users.T, items.T, W_user.T, W_item.T)

    return pl.pallas_call(
        _combine_body,
        out_shape=jax.ShapeDtypeStruct((_B,), jnp.float32),
    )(logits, mu_g, eps)


# speedup vs baseline: 1.0359x; 1.0067x over previous
import functools

import jax
import jax.numpy as jnp
from jax import lax
from jax.experimental import pallas as pl
from jax.experimental.pallas import tpu as pltpu
from jax.experimental.pallas import tpu_sc as plsc

_B = 16384
_F = 64

_NS = plsc.get_sparse_core_info().num_subcores
_NW = _NS
_BPW = _B // _NW
_CHUNK = 128
_NCHUNK = _BPW // _CHUNK


def _sc_gather(idx_hbm, mu_hbm, out_hbm, idx_v, mu_v, sem, wsem):
    wid = lax.axis_index("s")
    base = wid * _BPW
    pltpu.sync_copy(idx_hbm.at[pl.ds(wid * _NCHUNK, _NCHUNK)], idx_v)
    copies = [
        pltpu.async_copy(mu_hbm.at[idx_v.at[j]],
                         mu_v.at[pl.ds(j * _CHUNK, _CHUNK)], sem)
        for j in range(_NCHUNK)
    ]
    writes = []
    for j in range(_NCHUNK):
        copies[j].wait()
        writes.append(
            pltpu.async_copy(mu_v.at[pl.ds(j * _CHUNK, _CHUNK)],
                             out_hbm.at[pl.ds(base + j * _CHUNK, _CHUNK)],
                             wsem))
    for w in writes:
        w.wait()


def _matvec_body(ut_ref, vt_ref, wu_ref, wi_ref, o_ref):
    ut = ut_ref[...]
    vt = vt_ref[...]
    wu = wu_ref[...]
    wi = wi_ref[...]
    s = (jnp.dot(wu, ut, preferred_element_type=jnp.float32)
         + jnp.dot(wi, vt, preferred_element_type=jnp.float32))
    o_ref[...] = s.reshape(o_ref.shape)


def _combine_body(lg_ref, mu_ref, eps_ref, o_ref):
    o_ref[...] = lg_ref[...] + jnp.exp(mu_ref[...] + eps_ref[...])


def kernel(users, items, items_idx, eps, W_user, W_item, mu):
    idx2d = items_idx.reshape(_B // _CHUNK, _CHUNK)

    mesh = plsc.VectorSubcoreMesh(core_axis_name="c", subcore_axis_name="s",
                                  num_cores=1)
    sc = functools.partial(
        pl.kernel,
        mesh=mesh,
        out_type=jax.ShapeDtypeStruct((_B,), jnp.float32),
        scratch_types=[
            pltpu.VMEM((_NCHUNK, _CHUNK), jnp.int32),
            pltpu.VMEM((_BPW,), jnp.float32),
            pltpu.SemaphoreType.DMA,
            pltpu.SemaphoreType.DMA,
        ],
    )(_sc_gather)
    mu_g = sc(idx2d, mu)

    cols = 4096
    logits = pl.pallas_call(
        _matvec_body,
        grid=(_B // cols,),
        in_specs=[
            pl.BlockSpec((_F, cols), lambda i: (0, i)),
            pl.BlockSpec((_F, cols), lambda i: (0, i)),
            pl.BlockSpec((1, _F), lambda i: (0, 0)),
            pl.BlockSpec((1, _F), lambda i: (0, 0)),
        ],
        out_specs=pl.BlockSpec((cols,), lambda i: (i,)),
        out_shape=jax.ShapeDtypeStruct((_B,), jnp.float32),
    )(users.T, items.T, W_user.T, W_item.T)

    return pl.pallas_call(
        _combine_body,
        out_shape=jax.ShapeDtypeStruct((_B,), jnp.float32),
    )(logits, mu_g, eps)
